# Initial kernel scaffold; baseline (speedup 1.0000x reference)
import jax
import jax.numpy as jnp
from jax.experimental import pallas as pl


def kernel(features, lab_embeddings, prototype_spectra, score_w1, score_b1, score_w2, score_b2, ab_w1, ab_b1, ab_w2, ab_b2):
    raise NotImplementedError("write your pallas kernel here")


# speedup vs baseline: 4.0681x; 4.0681x over previous
import functools

import jax
import jax.numpy as jnp
from jax import lax
from jax.experimental import pallas as pl
from jax.experimental.pallas import tpu as pltpu
from jax.experimental.pallas import tpu_sc as plsc

EMBED = 768
HID = 512
K = 5
NLAB = 8192
SPEC = 256
N = 4096
BN = 128

NEG = float("-inf")


def _tc_body(x_ref, labT_ref, sw1_ref, sb1_ref, sw2_ref, sb2_ref,
             aw1x_ref, aw1s_ref, ab1_ref, aw2_ref, ab2_ref,
             cs_ref, ids_ref, ab_ref, dom_ref, wexp_ref):
    x = x_ref[...]

    s = jnp.dot(x, labT_ref[...], preferred_element_type=jnp.float32)
    col = lax.broadcasted_iota(jnp.int32, (BN, NLAB), 1)

    m_prev = jnp.full((BN, 1), jnp.inf, jnp.float32)
    i_prev = jnp.full((BN, 1), -1, jnp.int32)
    vals, idxs = [], []
    for _ in range(K):
        valid = (s < m_prev) | ((s == m_prev) & (col > i_prev))
        sv = jnp.where(valid, s, NEG)
        m = jnp.max(sv, axis=1, keepdims=True)
        idx = jnp.min(jnp.where(sv == m, col, NLAB), axis=1, keepdims=True)
        vals.append(m)
        idxs.append(idx)
        m_prev, i_prev = m, idx
    cand_sims = jnp.concatenate(vals, axis=1)
    cand_ids = jnp.concatenate(idxs, axis=1)

    h1 = jnp.maximum(
        jnp.dot(x, sw1_ref[...], preferred_element_type=jnp.float32)
        + sb1_ref[...], 0.0)
    scores = (jnp.dot(h1, sw2_ref[...], preferred_element_type=jnp.float32)
              + sb2_ref[...])

    h2 = jnp.maximum(
        jnp.dot(x, aw1x_ref[...], preferred_element_type=jnp.float32)
        + jnp.dot(scores, aw1s_ref[...], preferred_element_type=jnp.float32)
        + ab1_ref[...], 0.0)
    logits = (jnp.dot(h2, aw2_ref[...], preferred_element_type=jnp.float32)
              + ab2_ref[...])
    mx = jnp.max(logits, axis=1, keepdims=True)
    e = jnp.exp(logits - mx)
    ab = e / jnp.sum(e, axis=1, keepdims=True)

    ab5 = ab[:, :K]
    lane = lax.broadcasted_iota(jnp.int32, (BN, K), 1)
    am = jnp.max(ab5, axis=1, keepdims=True)
    slot = jnp.min(jnp.where(ab5 == am, lane, K), axis=1, keepdims=True)
    dom = jnp.max(jnp.where(lane == slot, cand_ids, -1), axis=1, keepdims=True)

    cs_ref[...] = cand_sims
    ids_ref[...] = cand_ids
    ab_ref[...] = ab
    dom_ref[...] = dom
    wexp_ref[...] = jnp.broadcast_to(ab5[:, :, None], (BN, K, 16)).reshape(BN, K * 16)


def _tc_head(flat, labT, sw1, sb1, sw2, sb2, aw1x, aw1s, ab1, aw2, ab2):
    nblocks = N // BN
    full = lambda r: (0, 0)
    grid_spec = pl.GridSpec(
        grid=(nblocks,),
        in_specs=[
            pl.BlockSpec((BN, EMBED), lambda r: (r, 0)),
            pl.BlockSpec((EMBED, NLAB), full),
            pl.BlockSpec((EMBED, HID), full),
            pl.BlockSpec((1, HID), full),
            pl.BlockSpec((HID, K), full),
            pl.BlockSpec((1, K), full),
            pl.BlockSpec((EMBED, HID), full),
            pl.BlockSpec((K, HID), full),
            pl.BlockSpec((1, HID), full),
            pl.BlockSpec((HID, K + 2), full),
            pl.BlockSpec((1, K + 2), full),
        ],
        out_specs=[
            pl.BlockSpec((BN, K), lambda r: (r, 0)),
            pl.BlockSpec((BN, K), lambda r: (r, 0)),
            pl.BlockSpec((BN, K + 2), lambda r: (r, 0)),
            pl.BlockSpec((BN, 1), lambda r: (r, 0)),
            pl.BlockSpec((BN, K * 16), lambda r: (r, 0)),
        ],
    )
    return pl.pallas_call(
        _tc_body,
        grid_spec=grid_spec,
        out_shape=[
            jax.ShapeDtypeStruct((N, K), jnp.float32),
            jax.ShapeDtypeStruct((N, K), jnp.int32),
            jax.ShapeDtypeStruct((N, K + 2), jnp.float32),
            jax.ShapeDtypeStruct((N, 1), jnp.int32),
            jax.ShapeDtypeStruct((N, K * 16), jnp.float32),
        ],
    )(flat, labT, sw1, sb1, sw2, sb2, aw1x, aw1s, ab1, aw2, ab2)


_NW = 32
_PPW = N // _NW
_CH = 16


def _sc_recon_body(ids_hbm, wexp_hbm, protos_hbm, out_hbm,
                   ids_v, w_v, rows_v, out_v, sem):
    c = lax.axis_index("c")
    s = lax.axis_index("s")
    wid = s * 2 + c
    base = wid * _PPW

    pltpu.sync_copy(ids_hbm.at[pl.ds(base * K, _PPW * K)], ids_v)

    for ch in range(_PPW // _CH):
        pltpu.sync_copy(
            wexp_hbm.at[pl.ds(base * K + ch * _CH * K, _CH * K)], w_v)
        pltpu.async_copy(
            protos_hbm.at[ids_v.at[pl.ds(ch * _CH * K, _CH * K)]],
            rows_v, sem).wait()

        def body(p, carry):
            fp = p * K
            ws = [w_v[fp + k, :] for k in range(K)]
            for c16 in range(SPEC // 16):
                sl = pl.ds(c16 * 16, 16)
                acc = ws[0] * rows_v[p * K, sl]
                for k in range(1, K):
                    acc = acc + ws[k] * rows_v[p * K + k, sl]
                out_v[p, sl] = acc
            return carry

        lax.fori_loop(0, _CH, body, 0)
        pltpu.sync_copy(out_v, out_hbm.at[pl.ds(base + ch * _CH, _CH)])


def _sc_recon(ids_flat, wexp, protos):
    mesh = plsc.VectorSubcoreMesh(core_axis_name="c", subcore_axis_name="s")
    f = functools.partial(
        pl.kernel,
        mesh=mesh,
        out_type=jax.ShapeDtypeStruct((N, SPEC), jnp.float32),
        scratch_types=[
            pltpu.VMEM((_PPW * K,), jnp.int32),
            pltpu.VMEM((_CH * K, 16), jnp.float32),
            pltpu.VMEM((_CH * K, SPEC), jnp.float32),
            pltpu.VMEM((_CH, SPEC), jnp.float32),
            pltpu.SemaphoreType.DMA,
        ],
    )(_sc_recon_body)
    return f(ids_flat, wexp, protos)


def kernel(features, lab_embeddings, prototype_spectra,
           score_w1, score_b1, score_w2, score_b2,
           ab_w1, ab_b1, ab_w2, ab_b2):
    b, h, w, cdim = features.shape
    flat = features.reshape(-1, cdim)
    labT = lab_embeddings.T
    aw1x = ab_w1[:EMBED]
    aw1s = ab_w1[EMBED:]

    cand_sims, cand_ids, abundances, dominant, wexp = _tc_head(
        flat, labT,
        score_w1, score_b1.reshape(1, -1), score_w2, score_b2.reshape(1, -1),
        aw1x, aw1s, ab_b1.reshape(1, -1), ab_w2, ab_b2.reshape(1, -1))

    ids_flat = cand_ids.reshape(-1)
    recon = _sc_recon(ids_flat, wexp.reshape(N * K, 16), prototype_spectra)

    return (dominant.reshape(b, h, w),
            abundances.reshape(b, h, w, -1),
            recon.reshape(b, h, w, -1),
            cand_sims.reshape(b, h, w, -1))
